# Initial kernel scaffold; baseline (speedup 1.0000x reference)
import jax
import jax.numpy as jnp
from jax.experimental import pallas as pl


def kernel(gx, logits, cur_token_ids):
    raise NotImplementedError("write your pallas kernel here")


# speedup vs baseline: 1.0062x; 1.0062x over previous
import jax
import jax.numpy as jnp
from jax import lax
from jax.experimental import pallas as pl

EPS_C = 1e-10
K_TOP = 250
INV_TEMP = 10.0


def _scale_body(x_ref, o_ref):
    o_ref[...] = x_ref[...] * INV_TEMP


def kernel(gx, logits, cur_token_ids):
    B, S, V = gx.shape
    _, ids = lax.top_k(logits, K_TOP)
    gxv = jnp.take_along_axis(gx, ids, axis=-1)
    dist = jnp.where(ids == cur_token_ids[..., None].astype(jnp.int32), EPS_C, 1.0)
    y = -(gxv * dist).reshape(B * S, K_TOP)
    out = pl.pallas_call(
        _scale_body,
        grid=(B * S // 8,),
        in_specs=[pl.BlockSpec((8, K_TOP), lambda i: (i, 0))],
        out_specs=pl.BlockSpec((8, K_TOP), lambda i: (i, 0)),
        out_shape=jax.ShapeDtypeStruct((B * S, K_TOP), jnp.float32),
    )(y)
    return out.reshape(B, S, K_TOP), ids


# speedup vs baseline: 11.0159x; 10.9478x over previous
import functools

import jax
import jax.numpy as jnp
from jax import lax
from jax.experimental import pallas as pl
from jax.experimental.pallas import tpu as pltpu
from jax.experimental.pallas import tpu_sc as plsc

EPS_C = 1e-10
K_TOP = 250
INV_TEMP = 10.0
NBINS = 16384
BIN_SHIFT = 18
CAP = 384
RB = 8


def _sc_stage1(V, n_rows_per_tile):
    nchunks = V // 16
    mesh = plsc.VectorSubcoreMesh(core_axis_name="c", subcore_axis_name="s")
    info = plsc.get_sparse_core_info()
    nc = info.num_cores

    def body(logits_hbm, gxflat_hbm, lo_hbm, scale_hbm,
             candv_hbm, candi_hbm, candg_hbm,
             row_v, hist_v, cv_v, ci_v, cg_v, lo_v, sc_v, sem):
        wid = lax.axis_index("s") * nc + lax.axis_index("c")
        base = wid * n_rows_per_tile
        lane = lax.iota(jnp.int32, 16)
        ones_i = jnp.ones((16,), jnp.int32)
        neginf = jnp.full((16,), -jnp.inf, jnp.float32)
        zero_i = jnp.zeros((16,), jnp.int32)

        def per_row(i, _):
            r = base + i
            pltpu.sync_copy(logits_hbm.at[r], row_v)

            def z_hist(c, _):
                hist_v[pl.ds(c * 16, 16)] = zero_i
                return ()
            lax.fori_loop(0, NBINS // 16, z_hist, ())

            def z_cand(c, _):
                cv_v[pl.ds(c * 16, 16)] = neginf
                ci_v[pl.ds(c * 16, 16)] = zero_i
                cg_v[pl.ds(c * 16, 16)] = jnp.zeros((16,), jnp.float32)
                return ()
            lax.fori_loop(0, CAP // 16, z_cand, ())

            pltpu.sync_copy(lo_hbm.at[r], lo_v)
            pltpu.sync_copy(scale_hbm.at[r], sc_v)
            m_lo = lo_v[...]
            scale = sc_v[...]

            def binify(v):
                b = ((v - m_lo) * scale).astype(jnp.int32)
                return jnp.minimum(b, jnp.int32(NBINS - 1))

            def hist_chunk(c, _):
                v = row_v[pl.ds(c * 16, 16)]
                plsc.addupdate_scatter(hist_v, [binify(v)], ones_i)
                return ()
            lax.fori_loop(0, nchunks, hist_chunk, ())

            def scan_chunk(c, carry):
                prefix, best = carry
                h = hist_v[pl.ds(c * 16, 16)]
                pre = plsc.cumsum(h)
                pexcl = pre - h + prefix
                q = pexcl <= jnp.int32(V - K_TOP)
                bins = c * 16 + lane
                cand = jnp.where(q, bins, jnp.int32(-1))
                best = jnp.maximum(best, jnp.max(cand))
                return prefix + jnp.sum(h), best
            _, bstar = lax.fori_loop(0, NBINS // 16, scan_chunk,
                                     (jnp.int32(0), jnp.int32(-1)))
            btv = jnp.full((16,), bstar, jnp.int32)

            def comp_chunk(c, off):
                v = row_v[pl.ds(c * 16, 16)]
                m = binify(v) >= btv
                mi = jnp.where(m, 1, 0).astype(jnp.int32)
                pos = plsc.cumsum(mi) - 1 + off
                m = m & (pos < jnp.int32(CAP))
                idx = c * 16 + lane
                plsc.store_scatter(cv_v, [pos], v, mask=m)
                plsc.store_scatter(ci_v, [pos], idx, mask=m)
                return off + jnp.sum(mi)
            lax.fori_loop(0, nchunks, comp_chunk, jnp.int32(0))

            rbase = r * V

            def gath(c, _):
                fidx = ci_v[pl.ds(c * 16, 16)] + rbase
                pltpu.async_copy(gxflat_hbm.at[fidx],
                                 cg_v.at[pl.ds(c * 16, 16)], sem).wait()
                return ()
            lax.fori_loop(0, CAP // 16, gath, ())

            pltpu.sync_copy(cv_v, candv_hbm.at[r])
            pltpu.sync_copy(ci_v, candi_hbm.at[r])
            pltpu.sync_copy(cg_v, candg_hbm.at[r])
            return ()

        lax.fori_loop(0, n_rows_per_tile, per_row, ())

    R = 32 * n_rows_per_tile
    return pl.kernel(
        body,
        mesh=mesh,
        out_type=[
            jax.ShapeDtypeStruct((R, CAP), jnp.float32),
            jax.ShapeDtypeStruct((R, CAP), jnp.int32),
            jax.ShapeDtypeStruct((R, CAP), jnp.float32),
        ],
        scratch_types=[
            pltpu.VMEM((V,), jnp.float32),
            pltpu.VMEM((NBINS,), jnp.int32),
            pltpu.VMEM((CAP,), jnp.float32),
            pltpu.VMEM((CAP,), jnp.int32),
            pltpu.VMEM((CAP,), jnp.float32),
            pltpu.VMEM((16,), jnp.float32),
            pltpu.VMEM((16,), jnp.float32),
            pltpu.SemaphoreType.DMA,
        ],
        compiler_params=pltpu.CompilerParams(needs_layout_passes=False),
    )


def _tc_minmax_body(lg_ref, lo_ref, sc_ref):
    x = lg_ref[...]
    lo = jnp.min(x, axis=1, keepdims=True)
    hi = jnp.max(x, axis=1, keepdims=True)
    scale = (NBINS - 1) / jnp.maximum(hi - lo, 1e-30)
    lo_ref[...] = jnp.broadcast_to(lo, lo_ref.shape)
    sc_ref[...] = jnp.broadcast_to(scale, sc_ref.shape)


def _tc_rank_body(cv_ref, ci_ref, cg_ref, cur_ref, op_ref, oi_ref):
    v = cv_ref[...]
    ix = ci_ref[...]
    g = cg_ref[...]
    vi = v[:, :, None]
    vj = v[:, None, :]
    ii = ix[:, :, None]
    ij = ix[:, None, :]
    beats = (vj > vi) | ((vj == vi) & (ij < ii))
    rank = jnp.sum(beats.astype(jnp.int32), axis=2)
    dist = jnp.where(ix == cur_ref[...], EPS_C, 1.0)
    y = -INV_TEMP * g * dist
    kio = lax.broadcasted_iota(jnp.int32, (RB, CAP, CAP), 2)
    e = rank[:, :, None] == kio
    op_ref[...] = jnp.sum(jnp.where(e, y[:, :, None], 0.0), axis=1)
    oi_ref[...] = jnp.sum(jnp.where(e, ix[:, :, None], 0), axis=1)


def kernel(gx, logits, cur_token_ids):
    B, S, V = gx.shape
    R = B * S
    lg2 = logits.reshape(R, V)
    gxflat = gx.reshape(R * V)
    cur = cur_token_ids.reshape(R, 1).astype(jnp.int32)

    lo_b, sc_b = pl.pallas_call(
        _tc_minmax_body,
        grid=(R // 8,),
        in_specs=[pl.BlockSpec((8, V), lambda i: (i, 0))],
        out_specs=[
            pl.BlockSpec((8, 16), lambda i: (i, 0)),
            pl.BlockSpec((8, 16), lambda i: (i, 0)),
        ],
        out_shape=[
            jax.ShapeDtypeStruct((R, 16), jnp.float32),
            jax.ShapeDtypeStruct((R, 16), jnp.float32),
        ],
    )(lg2)

    candv, candi, candg = _sc_stage1(V, R // 32)(lg2, gxflat, lo_b, sc_b)

    out_p, out_i = pl.pallas_call(
        _tc_rank_body,
        grid=(R // RB,),
        in_specs=[
            pl.BlockSpec((RB, CAP), lambda i: (i, 0)),
            pl.BlockSpec((RB, CAP), lambda i: (i, 0)),
            pl.BlockSpec((RB, CAP), lambda i: (i, 0)),
            pl.BlockSpec((RB, 1), lambda i: (i, 0)),
        ],
        out_specs=[
            pl.BlockSpec((RB, CAP), lambda i: (i, 0)),
            pl.BlockSpec((RB, CAP), lambda i: (i, 0)),
        ],
        out_shape=[
            jax.ShapeDtypeStruct((R, CAP), jnp.float32),
            jax.ShapeDtypeStruct((R, CAP), jnp.int32),
        ],
    )(candv, candi, candg, cur)
    return (out_p[:, :K_TOP].reshape(B, S, K_TOP),
            out_i[:, :K_TOP].reshape(B, S, K_TOP))
